# baseline (speedup 1.0000x reference)
import functools

import jax
import jax.numpy as jnp
from jax import lax
from jax.experimental import pallas as pl
from jax.experimental.pallas import tpu as pltpu
from jax.experimental.pallas import tpu_sc as plsc

NUM_ROWS = 400
EDIM = 32
XW = 16
OW = 78
PW = 80
CHUNK = 512
GSUB = 128


def _body(x_hbm, wpair_hbm, out_hbm, x_v, ip0, ip1, ip2, ip3, st_v, sem,
          *, tokens_per_worker, num_cores):
    ip_refs = (ip0, ip1, ip2, ip3)
    wid = lax.axis_index("s") * num_cores + lax.axis_index("c")
    base0 = wid * tokens_per_worker
    n_chunks = tokens_per_worker // CHUNK

    def do_chunk(ci, _):
        base = base0 + ci * CHUNK
        pltpu.sync_copy(x_hbm.at[pl.ds(base, CHUNK)], x_v)

        for k in range(CHUNK // GSUB):
            @plsc.parallel_loop(0, GSUB // 16, unroll=2)
            def extract(g, _k=k):
                tok = _k * GSUB + g * 16 + lax.iota(jnp.int32, 16)
                va = plsc.load_gather(x_v, [tok, jnp.zeros((16,), jnp.int32)])
                ia = jnp.clip(va.astype(jnp.int32), 0, NUM_ROWS - 1)
                vo = plsc.load_gather(x_v, [tok, jnp.full((16,), 8, jnp.int32)])
                io = jnp.clip(vo.astype(jnp.int32), 0, NUM_ROWS - 1)
                plsc.store_scatter(
                    ip_refs[_k], [g * 16 + lax.iota(jnp.int32, 16)],
                    ia * NUM_ROWS + io)

        copies = []
        for k in range(CHUNK // GSUB):
            copies.append(pltpu.make_async_copy(
                wpair_hbm.at[ip_refs[k]],
                st_v.at[pl.ds(k * GSUB, GSUB)], sem))
        for cp in copies:
            cp.start()
        for cp in copies:
            cp.wait()

        @plsc.parallel_loop(0, CHUNK // 16, unroll=2)
        def states(g):
            tok = g * 16 + lax.iota(jnp.int32, 16)

            def col(c):
                return jnp.full((16,), c, jnp.int32)

            for c in range(7):
                sa = plsc.load_gather(x_v, [tok, col(1 + c)])
                plsc.store_scatter(st_v, [tok, col(EDIM + c)], sa)
                so = plsc.load_gather(x_v, [tok, col(9 + c)])
                plsc.store_scatter(st_v, [tok, col(71 + c)], so)

        pltpu.sync_copy(st_v, out_hbm.at[pl.ds(base, CHUNK)])
        return 0

    lax.fori_loop(0, n_chunks, do_chunk, 0)


def kernel(x, W):
    B, H, _ = x.shape
    n = B * H
    mesh = plsc.VectorSubcoreMesh(core_axis_name="c", subcore_axis_name="s")
    n_workers = mesh.num_cores * mesh.num_subcores
    tokens_per_worker = n // n_workers
    assert tokens_per_worker * n_workers == n
    assert tokens_per_worker % CHUNK == 0 and CHUNK % GSUB == 0

    za = jnp.zeros((NUM_ROWS, NUM_ROWS, 7), W.dtype)
    wpair = jnp.concatenate(
        [
            jnp.broadcast_to(W[:, None, :], (NUM_ROWS, NUM_ROWS, EDIM)),
            za,
            jnp.broadcast_to(W[None, :, :], (NUM_ROWS, NUM_ROWS, EDIM)),
            jnp.zeros((NUM_ROWS, NUM_ROWS, PW - OW + 7), W.dtype),
        ],
        axis=-1,
    ).reshape(NUM_ROWS * NUM_ROWS, PW)

    body = functools.partial(
        _body,
        tokens_per_worker=tokens_per_worker,
        num_cores=mesh.num_cores,
    )
    out = pl.kernel(
        body,
        out_type=jax.ShapeDtypeStruct((n, PW), jnp.float32),
        mesh=mesh,
        compiler_params=pltpu.CompilerParams(
            needs_layout_passes=False,
            use_tc_tiling_on_sc=False,
            disable_bounds_checks=True,
        ),
        scratch_types=[
            pltpu.VMEM((CHUNK, XW), jnp.float32),
            pltpu.VMEM((GSUB,), jnp.int32),
            pltpu.VMEM((GSUB,), jnp.int32),
            pltpu.VMEM((GSUB,), jnp.int32),
            pltpu.VMEM((GSUB,), jnp.int32),
            pltpu.VMEM((CHUNK, PW), jnp.float32),
            pltpu.SemaphoreType.DMA,
        ],
    )(x.reshape(n, XW), wpair)
    return out[:, :OW].reshape(B, H, OW)


# speedup vs baseline: 8.7796x; 8.7796x over previous
import functools

import jax
import jax.numpy as jnp
from jax import lax
from jax.experimental import pallas as pl
from jax.experimental.pallas import tpu as pltpu
from jax.experimental.pallas import tpu_sc as plsc

NUM_ROWS = 400
EDIM = 32
XW = 16
OW = 78
CHUNK = 512
BATCH = 8


def _body(x_hbm, w_hbm, out_hbm, w_v, x_v, out_v, *, tokens_per_worker, num_cores):
    wid = lax.axis_index("s") * num_cores + lax.axis_index("c")
    pltpu.sync_copy(w_hbm, w_v)
    base0 = wid * tokens_per_worker
    n_chunks = tokens_per_worker // CHUNK

    def do_chunk(ci, _):
        base = base0 + ci * CHUNK
        pltpu.sync_copy(x_hbm.at[pl.ds(base, CHUNK)], x_v)

        @plsc.parallel_loop(0, CHUNK // 16, unroll=4)
        def do_group(g):
            tok = g * 16 + lax.iota(jnp.int32, 16)

            def col(c):
                return jnp.full((16,), c, jnp.int32)

            va = plsc.load_gather(x_v, [tok, col(0)])
            ia = jnp.clip(va.astype(jnp.int32), 0, NUM_ROWS - 1)
            vo = plsc.load_gather(x_v, [tok, col(8)])
            io = jnp.clip(vo.astype(jnp.int32), 0, NUM_ROWS - 1)

            plan = []
            for c in range(EDIM):
                plan.append((w_v, ia, c, c))
                plan.append((w_v, io, c, 39 + c))
            for c in range(7):
                plan.append((x_v, tok, 1 + c, EDIM + c))
                plan.append((x_v, tok, 9 + c, 71 + c))

            for i in range(0, len(plan), BATCH):
                batch = plan[i:i + BATCH]
                vals = [plsc.load_gather(ref, [rows, col(sc)])
                        for ref, rows, sc, _ in batch]
                for (_, _, _, dc), v in zip(batch, vals):
                    plsc.store_scatter(out_v, [tok, col(dc)], v)

        pltpu.sync_copy(out_v, out_hbm.at[pl.ds(base, CHUNK)])
        return 0

    lax.fori_loop(0, n_chunks, do_chunk, 0)


def kernel(x, W):
    B, H, _ = x.shape
    n = B * H
    mesh = plsc.VectorSubcoreMesh(core_axis_name="c", subcore_axis_name="s")
    n_workers = mesh.num_cores * mesh.num_subcores
    tokens_per_worker = n // n_workers
    assert tokens_per_worker * n_workers == n
    assert tokens_per_worker % CHUNK == 0

    body = functools.partial(
        _body,
        tokens_per_worker=tokens_per_worker,
        num_cores=mesh.num_cores,
    )
    out = pl.kernel(
        body,
        out_type=jax.ShapeDtypeStruct((n, OW), jnp.float32),
        mesh=mesh,
        compiler_params=pltpu.CompilerParams(
            needs_layout_passes=False,
            use_tc_tiling_on_sc=False,
            disable_bounds_checks=True,
        ),
        scratch_types=[
            pltpu.VMEM((NUM_ROWS, EDIM), jnp.float32),
            pltpu.VMEM((CHUNK, XW), jnp.float32),
            pltpu.VMEM((CHUNK, OW), jnp.float32),
        ],
    )(x.reshape(n, XW), W)
    return out.reshape(B, H, OW)


# speedup vs baseline: 8.9277x; 1.0169x over previous
import functools

import jax
import jax.numpy as jnp
from jax import lax
from jax.experimental import pallas as pl
from jax.experimental.pallas import tpu as pltpu
from jax.experimental.pallas import tpu_sc as plsc

NUM_ROWS = 400
EDIM = 32
XW = 16
OW = 78
CHUNK = 512
BATCH = 16


def _body(x_hbm, w_hbm, out_hbm, w_v, x_v, out_v, *, tokens_per_worker, num_cores):
    wid = lax.axis_index("s") * num_cores + lax.axis_index("c")
    pltpu.sync_copy(w_hbm, w_v)
    base0 = wid * tokens_per_worker
    n_chunks = tokens_per_worker // CHUNK

    def do_chunk(ci, _):
        base = base0 + ci * CHUNK
        pltpu.sync_copy(x_hbm.at[pl.ds(base, CHUNK)], x_v)

        @plsc.parallel_loop(0, CHUNK // 16, unroll=4)
        def do_group(g):
            tok = g * 16 + lax.iota(jnp.int32, 16)

            def col(c):
                return jnp.full((16,), c, jnp.int32)

            va = plsc.load_gather(x_v, [tok, col(0)])
            ia = jnp.clip(va.astype(jnp.int32), 0, NUM_ROWS - 1)
            vo = plsc.load_gather(x_v, [tok, col(8)])
            io = jnp.clip(vo.astype(jnp.int32), 0, NUM_ROWS - 1)

            plan = []
            for c in range(EDIM):
                plan.append((w_v, ia, c, c))
                plan.append((w_v, io, c, 39 + c))
            for c in range(7):
                plan.append((x_v, tok, 1 + c, EDIM + c))
                plan.append((x_v, tok, 9 + c, 71 + c))

            for i in range(0, len(plan), BATCH):
                batch = plan[i:i + BATCH]
                vals = [plsc.load_gather(ref, [rows, col(sc)])
                        for ref, rows, sc, _ in batch]
                for (_, _, _, dc), v in zip(batch, vals):
                    plsc.store_scatter(out_v, [tok, col(dc)], v)

        pltpu.sync_copy(out_v, out_hbm.at[pl.ds(base, CHUNK)])
        return 0

    lax.fori_loop(0, n_chunks, do_chunk, 0)


def kernel(x, W):
    B, H, _ = x.shape
    n = B * H
    mesh = plsc.VectorSubcoreMesh(core_axis_name="c", subcore_axis_name="s")
    n_workers = mesh.num_cores * mesh.num_subcores
    tokens_per_worker = n // n_workers
    assert tokens_per_worker * n_workers == n
    assert tokens_per_worker % CHUNK == 0

    body = functools.partial(
        _body,
        tokens_per_worker=tokens_per_worker,
        num_cores=mesh.num_cores,
    )
    out = pl.kernel(
        body,
        out_type=jax.ShapeDtypeStruct((n, OW), jnp.float32),
        mesh=mesh,
        compiler_params=pltpu.CompilerParams(
            needs_layout_passes=False,
            use_tc_tiling_on_sc=False,
            disable_bounds_checks=True,
        ),
        scratch_types=[
            pltpu.VMEM((NUM_ROWS, EDIM), jnp.float32),
            pltpu.VMEM((CHUNK, XW), jnp.float32),
            pltpu.VMEM((CHUNK, OW), jnp.float32),
        ],
    )(x.reshape(n, XW), W)
    return out.reshape(B, H, OW)


# speedup vs baseline: 15.3346x; 1.7176x over previous
import functools

import jax
import jax.numpy as jnp
from jax import lax
from jax.experimental import pallas as pl
from jax.experimental.pallas import tpu as pltpu
from jax.experimental.pallas import tpu_sc as plsc

NUM_ROWS = 400
EDIM = 32
XW = 16
WP = 33
OW = 78
CHUNK = 512
BATCH = 16


def _body(x_hbm, w_hbm, out_hbm, w_v, x_v0, x_v1, o_v0, o_v1,
          isem0, isem1, osem0, osem1, *, tokens_per_worker, num_cores):
    wid = lax.axis_index("s") * num_cores + lax.axis_index("c")
    pltpu.sync_copy(w_hbm, w_v)
    base0 = wid * tokens_per_worker
    n_chunks = tokens_per_worker // CHUNK
    x_bufs = (x_v0, x_v1)
    o_bufs = (o_v0, o_v1)
    i_sems = (isem0, isem1)
    o_sems = (osem0, osem1)

    def in_cp(c, b):
        return pltpu.make_async_copy(
            x_hbm.at[pl.ds(base0 + c * CHUNK, CHUNK)], x_bufs[b], i_sems[b])

    def out_cp(c, b):
        return pltpu.make_async_copy(
            o_bufs[b], out_hbm.at[pl.ds(base0 + c * CHUNK, CHUNK)], o_sems[b])

    def compute(x_v, out_v):
        @plsc.parallel_loop(0, CHUNK // 16, unroll=4)
        def do_group(g):
            tok = g * 16 + lax.iota(jnp.int32, 16)

            def col(c):
                return jnp.full((16,), c, jnp.int32)

            lane = lax.iota(jnp.int32, 16)
            va = plsc.load_gather(x_v, [tok, col(0)])
            ia = jnp.clip(va.astype(jnp.int32), 0, NUM_ROWS - 1)
            vo = plsc.load_gather(x_v, [tok, col(8)])
            io = jnp.clip(vo.astype(jnp.int32), 0, NUM_ROWS - 1)

            plan = []
            for c in range(EDIM):
                plan.append((ia, c, c))
                plan.append((io, c, 39 + c))
            for i in range(0, len(plan), BATCH):
                batch = plan[i:i + BATCH]
                vals = [plsc.load_gather(w_v, [rows, col(sc)])
                        for rows, sc, _ in batch]
                for (_, _, dc), v in zip(batch, vals):
                    plsc.store_scatter(out_v, [tok, col(dc)], v)

            for r in range(XW):
                src_c = jnp.bitwise_and(r + lane, XW - 1)
                v = plsc.load_gather(x_v, [tok, src_c])
                dst_c = jnp.where(src_c < 8, 31 + src_c, 62 + src_c)
                msk = jnp.logical_and(src_c != 0, src_c != 8)
                plsc.store_scatter(out_v, [tok, dst_c], v, mask=msk)

    in_cp(0, 0).start()
    in_cp(1, 1).start()

    def do_pair(ci2, _):
        for b in range(2):
            c = ci2 * 2 + b
            in_cp(c, b).wait()

            @pl.when(c >= 2)
            def _():
                out_cp(c - 2, b).wait()

            compute(x_bufs[b], o_bufs[b])
            out_cp(c, b).start()

            @pl.when(c + 2 < n_chunks)
            def _():
                in_cp(c + 2, b).start()
        return 0

    lax.fori_loop(0, n_chunks // 2, do_pair, 0)
    out_cp(n_chunks - 2, 0).wait()
    out_cp(n_chunks - 1, 1).wait()


def kernel(x, W):
    B, H, _ = x.shape
    n = B * H
    mesh = plsc.VectorSubcoreMesh(core_axis_name="c", subcore_axis_name="s")
    n_workers = mesh.num_cores * mesh.num_subcores
    tokens_per_worker = n // n_workers
    assert tokens_per_worker * n_workers == n
    assert tokens_per_worker % CHUNK == 0
    assert (tokens_per_worker // CHUNK) % 2 == 0

    body = functools.partial(
        _body,
        tokens_per_worker=tokens_per_worker,
        num_cores=mesh.num_cores,
    )
    out = pl.kernel(
        body,
        out_type=jax.ShapeDtypeStruct((n, OW), jnp.float32),
        mesh=mesh,
        compiler_params=pltpu.CompilerParams(
            needs_layout_passes=False,
            use_tc_tiling_on_sc=False,
            disable_bounds_checks=True,
        ),
        scratch_types=[
            pltpu.VMEM((NUM_ROWS, WP), jnp.float32),
            pltpu.VMEM((CHUNK, XW), jnp.float32),
            pltpu.VMEM((CHUNK, XW), jnp.float32),
            pltpu.VMEM((CHUNK, OW), jnp.float32),
            pltpu.VMEM((CHUNK, OW), jnp.float32),
            pltpu.SemaphoreType.DMA,
            pltpu.SemaphoreType.DMA,
            pltpu.SemaphoreType.DMA,
            pltpu.SemaphoreType.DMA,
        ],
    )(
        x.reshape(n, XW),
        jnp.pad(W, ((0, 0), (0, WP - EDIM))),
    )
    return out.reshape(B, H, OW)
